# baseline (speedup 1.0000x reference)
import functools

import jax
import jax.numpy as jnp
from jax.experimental import pallas as pl
from jax.experimental.pallas import tpu as pltpu

BR = 512
BC = 1024


def _vq_kernel(x_ref, v_ref, x2_ref, v2_ref, out_ref,
               val_lo, idx_lo, val_hi, idx_hi):
    j = pl.program_id(1)
    ncols = pl.num_programs(1)
    half = ncols // 2

    m = jax.lax.dot_general(
        x_ref[...], v_ref[...],
        dimension_numbers=(((1,), (1,)), ((), ())),
        preferred_element_type=jnp.float32,
    )
    d2 = (x2_ref[...] + v2_ref[...]) - 2.0 * m
    dist = jnp.sqrt(jnp.maximum(d2, 0.0))

    tile_min = jnp.min(dist, axis=1, keepdims=True)
    col = jax.lax.broadcasted_iota(jnp.int32, dist.shape, 1)
    big = jnp.int32(1 << 30)
    tile_idx = jnp.min(jnp.where(dist == tile_min, col, big), axis=1,
                       keepdims=True) + j * BC

    @pl.when(j == 0)
    def _init_lo():
        val_lo[...] = tile_min
        idx_lo[...] = tile_idx

    @pl.when((j > 0) & (j < half))
    def _update_lo():
        better = tile_min < val_lo[...]
        val_lo[...] = jnp.where(better, tile_min, val_lo[...])
        idx_lo[...] = jnp.where(better, tile_idx, idx_lo[...])

    @pl.when(j == half)
    def _init_hi():
        val_hi[...] = tile_min
        idx_hi[...] = tile_idx

    @pl.when(j > half)
    def _update_hi():
        better = tile_min < val_hi[...]
        val_hi[...] = jnp.where(better, tile_min, val_hi[...])
        idx_hi[...] = jnp.where(better, tile_idx, idx_hi[...])

    @pl.when(j == ncols - 1)
    def _emit():
        lo_rounded = val_lo[...].astype(jnp.bfloat16).astype(jnp.float32)
        take_hi = val_hi[...] < lo_rounded
        out_ref[...] = jnp.where(take_hi, idx_hi[...], idx_lo[...])


@functools.partial(jax.jit, static_argnames=())
def kernel(x, vectors):
    shape = x.shape[:-1]
    d = x.shape[-1]
    xf = x.reshape(-1, d)
    n = xf.shape[0]
    k = vectors.shape[0]

    x2 = jnp.sum(xf * xf, axis=-1, keepdims=True)
    v2 = jnp.sum(vectors * vectors, axis=-1)[None, :]

    xb = xf.astype(jnp.bfloat16)
    vb = vectors.astype(jnp.bfloat16)

    nr = n // BR
    nc = k // BC

    out = pl.pallas_call(
        _vq_kernel,
        grid=(nr, nc),
        in_specs=[
            pl.BlockSpec((BR, d), lambda i, j: (i, 0)),
            pl.BlockSpec((BC, d), lambda i, j: (j, 0)),
            pl.BlockSpec((BR, 1), lambda i, j: (i, 0)),
            pl.BlockSpec((1, BC), lambda i, j: (0, j)),
        ],
        out_specs=pl.BlockSpec((BR, 1), lambda i, j: (i, 0)),
        out_shape=jax.ShapeDtypeStruct((n, 1), jnp.int32),
        scratch_shapes=[
            pltpu.VMEM((BR, 1), jnp.float32),
            pltpu.VMEM((BR, 1), jnp.int32),
            pltpu.VMEM((BR, 1), jnp.float32),
            pltpu.VMEM((BR, 1), jnp.int32),
        ],
    )(xb, vb, x2, v2)

    return out.reshape(shape).astype(jnp.int64)


# speedup vs baseline: 1.0944x; 1.0944x over previous
import functools

import jax
import jax.numpy as jnp
from jax.experimental import pallas as pl
from jax.experimental.pallas import tpu as pltpu

BR = 512
BC = 1024


def _vq_kernel(x_ref, v_ref, x2_ref, v2_ref, col_ref, out_ref,
               val_lo, idx_lo, val_hi, idx_hi):
    j = pl.program_id(1)
    ncols = pl.num_programs(1)
    half = ncols // 2

    m = jax.lax.dot_general(
        x_ref[...], v_ref[...],
        dimension_numbers=(((1,), (1,)), ((), ())),
        preferred_element_type=jnp.float32,
    )
    s = x2_ref[...] + v2_ref[...]
    d2 = jnp.float32(-2.0) * m + s
    dist = jnp.sqrt(jnp.maximum(d2, 0.0))

    tile_min = jnp.min(dist, axis=1, keepdims=True)
    tile_idx = jnp.min(jnp.where(dist == tile_min, col_ref[...], jnp.inf),
                       axis=1, keepdims=True)

    @pl.when(j == 0)
    def _init_lo():
        val_lo[...] = tile_min
        idx_lo[...] = tile_idx

    @pl.when((j > 0) & (j < half))
    def _update_lo():
        better = tile_min < val_lo[...]
        val_lo[...] = jnp.where(better, tile_min, val_lo[...])
        idx_lo[...] = jnp.where(better, tile_idx, idx_lo[...])

    @pl.when(j == half)
    def _init_hi():
        val_hi[...] = tile_min
        idx_hi[...] = tile_idx

    @pl.when(j > half)
    def _update_hi():
        better = tile_min < val_hi[...]
        val_hi[...] = jnp.where(better, tile_min, val_hi[...])
        idx_hi[...] = jnp.where(better, tile_idx, idx_hi[...])

    @pl.when(j == ncols - 1)
    def _emit():
        lo_rounded = val_lo[...].astype(jnp.bfloat16).astype(jnp.float32)
        take_hi = val_hi[...] < lo_rounded
        best = jnp.where(take_hi, idx_hi[...], idx_lo[...])
        out_ref[...] = best.astype(jnp.int32)


@functools.partial(jax.jit, static_argnames=())
def kernel(x, vectors):
    shape = x.shape[:-1]
    d = x.shape[-1]
    xf = x.reshape(-1, d)
    n = xf.shape[0]
    k = vectors.shape[0]

    x2 = jnp.sum(xf * xf, axis=-1, keepdims=True)
    v2 = jnp.sum(vectors * vectors, axis=-1)[None, :]

    xb = xf.astype(jnp.bfloat16)
    vb = vectors.astype(jnp.bfloat16)
    colf = jnp.arange(k, dtype=jnp.float32)[None, :]

    nr = n // BR
    nc = k // BC

    out = pl.pallas_call(
        _vq_kernel,
        grid=(nr, nc),
        in_specs=[
            pl.BlockSpec((BR, d), lambda i, j: (i, 0)),
            pl.BlockSpec((BC, d), lambda i, j: (j, 0)),
            pl.BlockSpec((BR, 1), lambda i, j: (i, 0)),
            pl.BlockSpec((1, BC), lambda i, j: (0, j)),
            pl.BlockSpec((1, BC), lambda i, j: (0, j)),
        ],
        out_specs=pl.BlockSpec((BR, 1), lambda i, j: (i, 0)),
        out_shape=jax.ShapeDtypeStruct((n, 1), jnp.int32),
        scratch_shapes=[
            pltpu.VMEM((BR, 1), jnp.float32),
            pltpu.VMEM((BR, 1), jnp.float32),
            pltpu.VMEM((BR, 1), jnp.float32),
            pltpu.VMEM((BR, 1), jnp.float32),
        ],
    )(xb, vb, x2, v2, colf)

    return out.reshape(shape).astype(jnp.int64)


# speedup vs baseline: 1.1411x; 1.0427x over previous
import functools

import jax
import jax.numpy as jnp
from jax.experimental import pallas as pl
from jax.experimental.pallas import tpu as pltpu

BR = 512
BC = 1024
LANES = 128


def _vq_kernel(x_ref, v_ref, x2_ref, v2_ref, col_ref, out_ref,
               sv, sc, val_lo, idx_lo):
    j = pl.program_id(1)
    ncols = pl.num_programs(1)
    half = ncols // 2

    m = jax.lax.dot_general(
        x_ref[...], v_ref[...],
        dimension_numbers=(((1,), (1,)), ((), ())),
        preferred_element_type=jnp.float32,
    )
    s = x2_ref[...] + v2_ref[...]
    d2 = jnp.float32(-2.0) * m + s
    dist = jnp.sqrt(jnp.maximum(d2, 0.0))
    col = col_ref[...]

    @pl.when((j == 0) | (j == half))
    def _reset():
        sv[...] = jnp.full((BR, LANES), jnp.inf, jnp.float32)
        sc[...] = jnp.zeros((BR, LANES), jnp.float32)

    tv = dist[:, :LANES]
    tc = jnp.broadcast_to(col[:, :LANES], (BR, LANES))
    for c in range(1, BC // LANES):
        ch = dist[:, c * LANES:(c + 1) * LANES]
        cc = col[:, c * LANES:(c + 1) * LANES]
        better = ch < tv
        tv = jnp.where(better, ch, tv)
        tc = jnp.where(better, cc, tc)
    bet = tv < sv[...]
    sv[...] = jnp.where(bet, tv, sv[...])
    sc[...] = jnp.where(bet, tc, sc[...])

    def _resolve():
        v = sv[...]
        rv = jnp.min(v, axis=1, keepdims=True)
        ri = jnp.min(jnp.where(v == rv, sc[...], jnp.inf), axis=1,
                     keepdims=True)
        return rv, ri

    @pl.when(j == half - 1)
    def _save_lo():
        rv, ri = _resolve()
        val_lo[...] = rv
        idx_lo[...] = ri

    @pl.when(j == ncols - 1)
    def _emit():
        rv, ri = _resolve()
        lo_rounded = val_lo[...].astype(jnp.bfloat16).astype(jnp.float32)
        take_hi = rv < lo_rounded
        best = jnp.where(take_hi, ri, idx_lo[...])
        out_ref[...] = best.astype(jnp.int32)


@functools.partial(jax.jit, static_argnames=())
def kernel(x, vectors):
    shape = x.shape[:-1]
    d = x.shape[-1]
    xf = x.reshape(-1, d)
    n = xf.shape[0]
    k = vectors.shape[0]

    x2 = jnp.sum(xf * xf, axis=-1, keepdims=True)
    v2 = jnp.sum(vectors * vectors, axis=-1)[None, :]

    xb = xf.astype(jnp.bfloat16)
    vb = vectors.astype(jnp.bfloat16)
    colf = jnp.arange(k, dtype=jnp.float32)[None, :]

    nr = n // BR
    nc = k // BC

    out = pl.pallas_call(
        _vq_kernel,
        grid=(nr, nc),
        in_specs=[
            pl.BlockSpec((BR, d), lambda i, j: (i, 0)),
            pl.BlockSpec((BC, d), lambda i, j: (j, 0)),
            pl.BlockSpec((BR, 1), lambda i, j: (i, 0)),
            pl.BlockSpec((1, BC), lambda i, j: (0, j)),
            pl.BlockSpec((1, BC), lambda i, j: (0, j)),
        ],
        out_specs=pl.BlockSpec((BR, 1), lambda i, j: (i, 0)),
        out_shape=jax.ShapeDtypeStruct((n, 1), jnp.int32),
        scratch_shapes=[
            pltpu.VMEM((BR, LANES), jnp.float32),
            pltpu.VMEM((BR, LANES), jnp.float32),
            pltpu.VMEM((BR, 1), jnp.float32),
            pltpu.VMEM((BR, 1), jnp.float32),
        ],
    )(xb, vb, x2, v2, colf)

    return out.reshape(shape).astype(jnp.int64)
